# baseline (speedup 1.0000x reference)
import functools

import jax
import jax.numpy as jnp
from jax import lax
from jax.experimental import pallas as pl
from jax.experimental.pallas import tpu as pltpu
from jax.experimental.pallas import tpu_sc as plsc

N_C = 5000
N_P = 5000
D_IN = 128
D_H = 32
D_OUT = 32
D_E = 16
D_EH = 32
E = 80000
P = 10000

E_PAD = 81920
N_PAD = 5120
P_PAD = 10240
DS = 40
EB = 2048

_NCORES = 2
_NSUB = 16
_NW = _NCORES * _NSUB
_EPW = E_PAD // _NW
_EPS = E_PAD // _NCORES
_EPT = _EPS // _NSUB
_NPT = N_PAD // _NSUB
_PPT = P_PAD // _NW


def _sc_mesh():
    return plsc.VectorSubcoreMesh(core_axis_name="c", subcore_axis_name="s")


_SC_PARAMS = pltpu.CompilerParams(use_tc_tiling_on_sc=False)


def _proj(hcst, hprd, Wu, bu, Wi, bi):
    def body(a_ref, b_ref, wu_ref, bu_ref, wi_ref, bi_ref, oc_ref, op_ref):
        oc_ref[...] = (jnp.dot(a_ref[...], wu_ref[...],
                               preferred_element_type=jnp.float32)
                       + bu_ref[...]).astype(jnp.bfloat16)
        op_ref[...] = (jnp.dot(b_ref[...], wi_ref[...],
                               preferred_element_type=jnp.float32)
                       + bi_ref[...]).astype(jnp.bfloat16)

    return pl.pallas_call(
        body,
        out_shape=[jax.ShapeDtypeStruct((N_C, D_H), jnp.bfloat16),
                   jax.ShapeDtypeStruct((N_P, D_H), jnp.bfloat16)],
    )(hcst, hprd, Wu, bu.reshape(1, D_H), Wi, bi.reshape(1, D_H))


def _gather_pipe(tasks, ch, idx_h, out_h, idx_v, rows_v, gsem, wsem):
    n = len(tasks)
    wb = [None, None]
    tab0, base0 = tasks[0]
    pltpu.sync_copy(idx_h.at[pl.ds(base0, ch)], idx_v[0])
    g_prev = pltpu.async_copy(tab0.at[idx_v[0]], rows_v[0], gsem[0])
    for j in range(n):
        b = j % 2
        nxt = (j + 1) % 2
        if j + 1 < n:
            if wb[nxt] is not None:
                wb[nxt].wait()
                wb[nxt] = None
            tab1, base1 = tasks[j + 1]
            pltpu.sync_copy(idx_h.at[pl.ds(base1, ch)], idx_v[nxt])
            g_next = pltpu.async_copy(tab1.at[idx_v[nxt]], rows_v[nxt],
                                      gsem[nxt])
        g_prev.wait()
        wb[b] = pltpu.async_copy(rows_v[b], out_h.at[pl.ds(tasks[j][1], ch)],
                                 wsem[b])
        if j + 1 < n:
            g_prev = g_next
    for w in wb:
        if w is not None:
            w.wait()


def _gather_src(hc, hp, src_all):
    ch = _EPW // 2

    @functools.partial(
        pl.kernel,
        out_type=jax.ShapeDtypeStruct((2 * E_PAD, D_H), jnp.bfloat16),
        mesh=_sc_mesh(),
        compiler_params=_SC_PARAMS,
        scratch_types=[pltpu.VMEM((ch,), jnp.int32),
                       pltpu.VMEM((ch,), jnp.int32),
                       pltpu.VMEM((ch, D_H), jnp.bfloat16),
                       pltpu.VMEM((ch, D_H), jnp.bfloat16),
                       pltpu.SemaphoreType.DMA,
                       pltpu.SemaphoreType.DMA,
                       pltpu.SemaphoreType.DMA,
                       pltpu.SemaphoreType.DMA],
    )
    def k(hc_h, hp_h, idx_h, out_h, i0, i1, r0, r1, g0, g1, w0, w1):
        wid = lax.axis_index("s") * _NCORES + lax.axis_index("c")
        tasks = []
        for t, tab in enumerate((hc_h, hp_h)):
            for c in range(2):
                tasks.append((tab, t * E_PAD + wid * _EPW + c * ch))
        _gather_pipe(tasks, ch, idx_h, out_h,
                     (i0, i1), (r0, r1), (g0, g1), (w0, w1))

    return k(hc, hp, src_all)


def _edge(ef_all, src_all, W1Ks, b1rs, W2ps, B2s, EH):
    nb = E_PAD // EB

    def body(ef_ref, src_ref, w1k_ref, b1r_ref, w2_ref, bb_ref,
             eh_ref, out_ref):
        i = pl.program_id(0)
        src16 = src_ref[...]
        hid_rep = jnp.maximum(
            jnp.dot(ef_ref[...], w1k_ref[0],
                    preferred_element_type=jnp.float32) + b1r_ref[0], 0.0)
        G = hid_rep * jnp.dot(src16, eh_ref[...],
                              preferred_element_type=jnp.float32)
        msg = (jnp.dot(G, w2_ref[0], preferred_element_type=jnp.float32)
               + jnp.dot(src16.astype(jnp.float32), bb_ref[0],
                         preferred_element_type=jnp.float32))
        rows = lax.rem(i, nb) * EB + lax.broadcasted_iota(jnp.int32, (EB, 1), 0)
        maskf = (rows < E).astype(jnp.float32)
        extra = maskf * (lax.broadcasted_iota(jnp.int32, (EB, 8), 1) == 0
                         ).astype(jnp.float32)
        out_ref[...] = jnp.concatenate([msg * maskf, extra], axis=1)

    return pl.pallas_call(
        body,
        grid=(2 * nb,),
        in_specs=[pl.BlockSpec((EB, D_E), lambda i: (i, 0)),
                  pl.BlockSpec((EB, D_H), lambda i: (i, 0)),
                  pl.BlockSpec((1, D_E, D_EH * D_H), lambda i: (i // nb, 0, 0)),
                  pl.BlockSpec((1, 1, D_EH * D_H), lambda i: (i // nb, 0, 0)),
                  pl.BlockSpec((1, D_EH * D_H, D_OUT), lambda i: (i // nb, 0, 0)),
                  pl.BlockSpec((1, D_H, D_OUT), lambda i: (i // nb, 0, 0)),
                  pl.BlockSpec((D_H, D_EH * D_H), lambda i: (0, 0))],
        out_specs=pl.BlockSpec((EB, DS), lambda i: (i, 0)),
        out_shape=jax.ShapeDtypeStruct((2 * E_PAD, DS), jnp.float32),
    )(ef_all, src_all, W1Ks, b1rs, W2ps, B2s, EH)


def _scatter(msg_all, dst_all, zeros_nd):
    @functools.partial(
        pl.kernel,
        out_type=(jax.ShapeDtypeStruct((_NCORES, N_PAD, DS), jnp.float32),
                  jax.ShapeDtypeStruct((_NCORES, N_PAD, DS), jnp.float32)),
        mesh=_sc_mesh(),
        compiler_params=_SC_PARAMS,
        scratch_types=[pltpu.VMEM((_EPT,), jnp.int32),
                       pltpu.VMEM((_EPT, DS), jnp.float32),
                       pltpu.VMEM_SHARED((N_PAD, DS), jnp.float32),
                       pltpu.VMEM_SHARED((N_PAD, DS), jnp.float32),
                       pltpu.SemaphoreType.DMA],
    )
    def k(m_h, d_h, z_h, go_h, gr_h, idx_v, msg_v, acc_o, acc_r, sem):
        c = lax.axis_index("c")
        s = lax.axis_index("s")
        r0 = s * _NPT
        pltpu.sync_copy(z_h.at[pl.ds(r0, _NPT)], acc_o.at[pl.ds(r0, _NPT)])
        pltpu.sync_copy(z_h.at[pl.ds(r0, _NPT)], acc_r.at[pl.ds(r0, _NPT)])
        plsc.subcore_barrier()
        for t, acc in enumerate((acc_o, acc_r)):
            base = t * E_PAD + c * _EPS + s * _EPT
            pltpu.sync_copy(d_h.at[pl.ds(base, _EPT)], idx_v)
            pltpu.sync_copy(m_h.at[pl.ds(base, _EPT)], msg_v)
            pltpu.sync_copy(msg_v, acc.at[idx_v], add=True)
        plsc.subcore_barrier()
        pltpu.sync_copy(acc_o.at[pl.ds(r0, _NPT)], go_h.at[c, pl.ds(r0, _NPT)])
        pltpu.sync_copy(acc_r.at[pl.ds(r0, _NPT)], gr_h.at[c, pl.ds(r0, _NPT)])

    return k(msg_all, dst_all, zeros_nd)


def _finish(agg_r, agg_o):
    def body(ar_ref, ao_ref, hc_ref, hp_ref):
        for a_ref, o_ref in ((ar_ref, hc_ref), (ao_ref, hp_ref)):
            a = a_ref[0, :N_C, :] + a_ref[1, :N_C, :]
            mean = a[:, :D_OUT] / jnp.maximum(a[:, D_OUT:D_OUT + 1], 1.0)
            nrm = jnp.sqrt(jnp.sum(mean * mean, axis=1, keepdims=True))
            o_ref[...] = mean / (nrm + 1e-12)

    return pl.pallas_call(
        body,
        out_shape=[jax.ShapeDtypeStruct((N_C, D_OUT), jnp.float32),
                   jax.ShapeDtypeStruct((N_P, D_OUT), jnp.float32)],
    )(agg_r, agg_o)


def _gather_cos(hc_new, hp_new, idx4):
    @functools.partial(
        pl.kernel,
        out_type=jax.ShapeDtypeStruct((4 * P_PAD, D_OUT), jnp.float32),
        mesh=_sc_mesh(),
        compiler_params=_SC_PARAMS,
        scratch_types=[pltpu.VMEM((_PPT,), jnp.int32),
                       pltpu.VMEM((_PPT,), jnp.int32),
                       pltpu.VMEM((_PPT, D_OUT), jnp.float32),
                       pltpu.VMEM((_PPT, D_OUT), jnp.float32),
                       pltpu.SemaphoreType.DMA,
                       pltpu.SemaphoreType.DMA,
                       pltpu.SemaphoreType.DMA,
                       pltpu.SemaphoreType.DMA],
    )
    def k(hcn_h, hpn_h, idx_h, out_h, i0, i1, r0, r1, g0, g1, w0, w1):
        wid = lax.axis_index("s") * _NCORES + lax.axis_index("c")
        tasks = [(tab, g * P_PAD + wid * _PPT)
                 for g, tab in enumerate((hcn_h, hpn_h, hcn_h, hpn_h))]
        _gather_pipe(tasks, _PPT, idx_h, out_h,
                     (i0, i1), (r0, r1), (g0, g1), (w0, w1))

    return k(hc_new, hp_new, idx4)


def _cos(rows4):
    ones = jnp.ones((D_OUT, 1), jnp.float32)

    def body(r_ref, o_ref, p_ref, n_ref):
        def cs(x, y):
            num = jnp.dot(x * y, o_ref[...], preferred_element_type=jnp.float32)
            nx = jnp.dot(x * x, o_ref[...], preferred_element_type=jnp.float32)
            ny = jnp.dot(y * y, o_ref[...], preferred_element_type=jnp.float32)
            return num / (jnp.sqrt(nx) * jnp.sqrt(ny) + 1e-12)

        p_ref[...] = cs(r_ref[0], r_ref[1])[:P]
        n_ref[...] = cs(r_ref[2], r_ref[3])[:P]

    return pl.pallas_call(
        body,
        out_shape=[jax.ShapeDtypeStruct((P, 1), jnp.float32),
                   jax.ShapeDtypeStruct((P, 1), jnp.float32)],
    )(rows4, ones)


def _pad1(x, n):
    return jnp.concatenate([x, jnp.zeros((n - x.shape[0],), x.dtype)])


def kernel(h_customer, h_product, edge_feat_orders, edge_feat_rev,
           W_user, b_user, W_item, b_item,
           W1_o, b1_o, W2_o, b2_o, W1_r, b1_r, W2_r, b2_r,
           edge_index_orders, edge_index_rev, pos_edge_index, neg_edge_index):
    hc, hp = _proj(h_customer, h_product, W_user, b_user, W_item, b_item)

    src_all = jnp.concatenate([_pad1(edge_index_orders[0], E_PAD),
                               _pad1(edge_index_rev[0], E_PAD)])
    dst_all = jnp.concatenate([_pad1(edge_index_orders[1], E_PAD),
                               _pad1(edge_index_rev[1], E_PAD)])

    src_rows = _gather_src(hc, hp, src_all)

    pad_ef = jnp.zeros((E_PAD - E, D_E), jnp.float32)
    ef_all = jnp.concatenate([edge_feat_orders, pad_ef,
                              edge_feat_rev, pad_ef])
    col = jnp.arange(D_EH * D_H, dtype=jnp.int32)[None, :]
    EK = (col // D_H == jnp.arange(D_EH, dtype=jnp.int32)[:, None]
          ).astype(jnp.float32)
    EH = (col % D_H == jnp.arange(D_H, dtype=jnp.int32)[:, None]
          ).astype(jnp.float32)
    W1Ks = jnp.stack([W1_o @ EK, W1_r @ EK])
    b1rs = jnp.stack([jnp.repeat(b1_o, D_H).reshape(1, D_EH * D_H),
                      jnp.repeat(b1_r, D_H).reshape(1, D_EH * D_H)])
    W2ps = jnp.stack([W2_o.reshape(D_EH * D_H, D_OUT),
                      W2_r.reshape(D_EH * D_H, D_OUT)])
    B2s = jnp.stack([b2_o.reshape(D_H, D_OUT), b2_r.reshape(D_H, D_OUT)])

    msg_all = _edge(ef_all, src_rows, W1Ks, b1rs, W2ps, B2s,
                    EH.astype(jnp.bfloat16))

    zeros_nd = jnp.zeros((N_PAD, DS), jnp.float32)
    agg_o, agg_r = _scatter(msg_all, dst_all, zeros_nd)

    hc_new, hp_new = _finish(agg_r, agg_o)

    idx4 = jnp.concatenate([_pad1(pos_edge_index[0], P_PAD),
                            _pad1(pos_edge_index[1], P_PAD),
                            _pad1(neg_edge_index[0], P_PAD),
                            _pad1(neg_edge_index[1], P_PAD)])
    rows4 = _gather_cos(hc_new, hp_new, idx4).reshape(4, P_PAD, D_OUT)
    pos_score, neg_score = _cos(rows4)

    return hc_new, hp_new, pos_score.reshape(P), neg_score.reshape(P)


# speedup vs baseline: 1.0084x; 1.0084x over previous
import functools

import jax
import jax.numpy as jnp
from jax import lax
from jax.experimental import pallas as pl
from jax.experimental.pallas import tpu as pltpu
from jax.experimental.pallas import tpu_sc as plsc

N_C = 5000
N_P = 5000
D_IN = 128
D_H = 32
D_OUT = 32
D_E = 16
D_EH = 32
E = 80000
P = 10000

E_PAD = 81920
N_PAD = 5120
P_PAD = 10240
DS = 40
EB = 4096

_NCORES = 2
_NSUB = 16
_NW = _NCORES * _NSUB
_EPW = E_PAD // _NW
_EPS = E_PAD // _NCORES
_EPT = _EPS // _NSUB
_NPT = N_PAD // _NSUB
_PPT = P_PAD // _NW


def _sc_mesh():
    return plsc.VectorSubcoreMesh(core_axis_name="c", subcore_axis_name="s")


_SC_PARAMS = pltpu.CompilerParams(use_tc_tiling_on_sc=False)


def _proj(hcst, hprd, Wu, bu, Wi, bi):
    def body(a_ref, b_ref, wu_ref, bu_ref, wi_ref, bi_ref, oc_ref, op_ref):
        oc_ref[...] = (jnp.dot(a_ref[...], wu_ref[...],
                               preferred_element_type=jnp.float32)
                       + bu_ref[...]).astype(jnp.bfloat16)
        op_ref[...] = (jnp.dot(b_ref[...], wi_ref[...],
                               preferred_element_type=jnp.float32)
                       + bi_ref[...]).astype(jnp.bfloat16)

    return pl.pallas_call(
        body,
        out_shape=[jax.ShapeDtypeStruct((N_C, D_H), jnp.bfloat16),
                   jax.ShapeDtypeStruct((N_P, D_H), jnp.bfloat16)],
    )(hcst, hprd, Wu, bu.reshape(1, D_H), Wi, bi.reshape(1, D_H))


def _gather_pipe(tasks, ch, idx_h, out_h, idx_v, rows_v, gsem, wsem):
    n = len(tasks)
    wb = [None, None]
    tab0, base0 = tasks[0]
    pltpu.sync_copy(idx_h.at[pl.ds(base0, ch)], idx_v[0])
    g_prev = pltpu.async_copy(tab0.at[idx_v[0]], rows_v[0], gsem[0])
    for j in range(n):
        b = j % 2
        nxt = (j + 1) % 2
        if j + 1 < n:
            if wb[nxt] is not None:
                wb[nxt].wait()
                wb[nxt] = None
            tab1, base1 = tasks[j + 1]
            pltpu.sync_copy(idx_h.at[pl.ds(base1, ch)], idx_v[nxt])
            g_next = pltpu.async_copy(tab1.at[idx_v[nxt]], rows_v[nxt],
                                      gsem[nxt])
        g_prev.wait()
        wb[b] = pltpu.async_copy(rows_v[b], out_h.at[pl.ds(tasks[j][1], ch)],
                                 wsem[b])
        if j + 1 < n:
            g_prev = g_next
    for w in wb:
        if w is not None:
            w.wait()


def _gather_src(hc, hp, src_all):
    ch = _EPW // 2

    @functools.partial(
        pl.kernel,
        out_type=jax.ShapeDtypeStruct((2 * E_PAD, D_H), jnp.bfloat16),
        mesh=_sc_mesh(),
        compiler_params=_SC_PARAMS,
        scratch_types=[pltpu.VMEM((ch,), jnp.int32),
                       pltpu.VMEM((ch,), jnp.int32),
                       pltpu.VMEM((ch, D_H), jnp.bfloat16),
                       pltpu.VMEM((ch, D_H), jnp.bfloat16),
                       pltpu.SemaphoreType.DMA,
                       pltpu.SemaphoreType.DMA,
                       pltpu.SemaphoreType.DMA,
                       pltpu.SemaphoreType.DMA],
    )
    def k(hc_h, hp_h, idx_h, out_h, i0, i1, r0, r1, g0, g1, w0, w1):
        wid = lax.axis_index("s") * _NCORES + lax.axis_index("c")
        tasks = []
        for t, tab in enumerate((hc_h, hp_h)):
            for c in range(2):
                tasks.append((tab, t * E_PAD + wid * _EPW + c * ch))
        _gather_pipe(tasks, ch, idx_h, out_h,
                     (i0, i1), (r0, r1), (g0, g1), (w0, w1))

    return k(hc, hp, src_all)


def _edge(ef_all, src_all, W1Ks, b1rs, W2ps, B2s, EH):
    nb = E_PAD // EB

    def body(ef_ref, src_ref, w1k_ref, b1r_ref, w2_ref, bb_ref,
             eh_ref, out_ref):
        i = pl.program_id(0)
        src16 = src_ref[...]
        hid_rep = jnp.maximum(
            jnp.dot(ef_ref[...], w1k_ref[0],
                    preferred_element_type=jnp.float32) + b1r_ref[0], 0.0)
        G = hid_rep * jnp.dot(src16, eh_ref[...],
                              preferred_element_type=jnp.float32)
        msg = (jnp.dot(G, w2_ref[0], preferred_element_type=jnp.float32)
               + jnp.dot(src16.astype(jnp.float32), bb_ref[0],
                         preferred_element_type=jnp.float32))
        rows = lax.rem(i, nb) * EB + lax.broadcasted_iota(jnp.int32, (EB, 1), 0)
        maskf = (rows < E).astype(jnp.float32)
        extra = maskf * (lax.broadcasted_iota(jnp.int32, (EB, 8), 1) == 0
                         ).astype(jnp.float32)
        out_ref[...] = jnp.concatenate([msg * maskf, extra], axis=1)

    return pl.pallas_call(
        body,
        grid=(2 * nb,),
        in_specs=[pl.BlockSpec((EB, D_E), lambda i: (i, 0)),
                  pl.BlockSpec((EB, D_H), lambda i: (i, 0)),
                  pl.BlockSpec((1, D_E, D_EH * D_H), lambda i: (i // nb, 0, 0)),
                  pl.BlockSpec((1, 1, D_EH * D_H), lambda i: (i // nb, 0, 0)),
                  pl.BlockSpec((1, D_EH * D_H, D_OUT), lambda i: (i // nb, 0, 0)),
                  pl.BlockSpec((1, D_H, D_OUT), lambda i: (i // nb, 0, 0)),
                  pl.BlockSpec((D_H, D_EH * D_H), lambda i: (0, 0))],
        out_specs=pl.BlockSpec((EB, DS), lambda i: (i, 0)),
        out_shape=jax.ShapeDtypeStruct((2 * E_PAD, DS), jnp.float32),
    )(ef_all, src_all, W1Ks, b1rs, W2ps, B2s, EH)


def _scatter(msg_all, dst_all, zeros_nd):
    @functools.partial(
        pl.kernel,
        out_type=(jax.ShapeDtypeStruct((_NCORES, N_PAD, DS), jnp.float32),
                  jax.ShapeDtypeStruct((_NCORES, N_PAD, DS), jnp.float32)),
        mesh=_sc_mesh(),
        compiler_params=_SC_PARAMS,
        scratch_types=[pltpu.VMEM((_EPT,), jnp.int32),
                       pltpu.VMEM((_EPT, DS), jnp.float32),
                       pltpu.VMEM_SHARED((N_PAD, DS), jnp.float32),
                       pltpu.VMEM_SHARED((N_PAD, DS), jnp.float32),
                       pltpu.SemaphoreType.DMA],
    )
    def k(m_h, d_h, z_h, go_h, gr_h, idx_v, msg_v, acc_o, acc_r, sem):
        c = lax.axis_index("c")
        s = lax.axis_index("s")
        r0 = s * _NPT
        pltpu.sync_copy(z_h.at[pl.ds(r0, _NPT)], acc_o.at[pl.ds(r0, _NPT)])
        pltpu.sync_copy(z_h.at[pl.ds(r0, _NPT)], acc_r.at[pl.ds(r0, _NPT)])
        plsc.subcore_barrier()
        for t, acc in enumerate((acc_o, acc_r)):
            base = t * E_PAD + c * _EPS + s * _EPT
            pltpu.sync_copy(d_h.at[pl.ds(base, _EPT)], idx_v)
            pltpu.sync_copy(m_h.at[pl.ds(base, _EPT)], msg_v)
            pltpu.sync_copy(msg_v, acc.at[idx_v], add=True)
        plsc.subcore_barrier()
        pltpu.sync_copy(acc_o.at[pl.ds(r0, _NPT)], go_h.at[c, pl.ds(r0, _NPT)])
        pltpu.sync_copy(acc_r.at[pl.ds(r0, _NPT)], gr_h.at[c, pl.ds(r0, _NPT)])

    return k(msg_all, dst_all, zeros_nd)


def _finish(agg_r, agg_o):
    def body(ar_ref, ao_ref, hc_ref, hp_ref):
        for a_ref, o_ref in ((ar_ref, hc_ref), (ao_ref, hp_ref)):
            a = a_ref[0, :N_C, :] + a_ref[1, :N_C, :]
            mean = a[:, :D_OUT] / jnp.maximum(a[:, D_OUT:D_OUT + 1], 1.0)
            nrm = jnp.sqrt(jnp.sum(mean * mean, axis=1, keepdims=True))
            o_ref[...] = mean / (nrm + 1e-12)

    return pl.pallas_call(
        body,
        out_shape=[jax.ShapeDtypeStruct((N_C, D_OUT), jnp.float32),
                   jax.ShapeDtypeStruct((N_P, D_OUT), jnp.float32)],
    )(agg_r, agg_o)


def _gather_cos(hc_new, hp_new, idx4):
    @functools.partial(
        pl.kernel,
        out_type=jax.ShapeDtypeStruct((4 * P_PAD, D_OUT), jnp.float32),
        mesh=_sc_mesh(),
        compiler_params=_SC_PARAMS,
        scratch_types=[pltpu.VMEM((_PPT,), jnp.int32),
                       pltpu.VMEM((_PPT,), jnp.int32),
                       pltpu.VMEM((_PPT, D_OUT), jnp.float32),
                       pltpu.VMEM((_PPT, D_OUT), jnp.float32),
                       pltpu.SemaphoreType.DMA,
                       pltpu.SemaphoreType.DMA,
                       pltpu.SemaphoreType.DMA,
                       pltpu.SemaphoreType.DMA],
    )
    def k(hcn_h, hpn_h, idx_h, out_h, i0, i1, r0, r1, g0, g1, w0, w1):
        wid = lax.axis_index("s") * _NCORES + lax.axis_index("c")
        tasks = [(tab, g * P_PAD + wid * _PPT)
                 for g, tab in enumerate((hcn_h, hpn_h, hcn_h, hpn_h))]
        _gather_pipe(tasks, _PPT, idx_h, out_h,
                     (i0, i1), (r0, r1), (g0, g1), (w0, w1))

    return k(hc_new, hp_new, idx4)


def _cos(rows4):
    ones = jnp.ones((D_OUT, 1), jnp.float32)

    def body(r_ref, o_ref, p_ref, n_ref):
        def cs(x, y):
            num = jnp.dot(x * y, o_ref[...], preferred_element_type=jnp.float32)
            nx = jnp.dot(x * x, o_ref[...], preferred_element_type=jnp.float32)
            ny = jnp.dot(y * y, o_ref[...], preferred_element_type=jnp.float32)
            return num / (jnp.sqrt(nx) * jnp.sqrt(ny) + 1e-12)

        p_ref[...] = cs(r_ref[0], r_ref[1])[:P]
        n_ref[...] = cs(r_ref[2], r_ref[3])[:P]

    return pl.pallas_call(
        body,
        out_shape=[jax.ShapeDtypeStruct((P, 1), jnp.float32),
                   jax.ShapeDtypeStruct((P, 1), jnp.float32)],
    )(rows4, ones)


def _pad1(x, n):
    return jnp.concatenate([x, jnp.zeros((n - x.shape[0],), x.dtype)])


def kernel(h_customer, h_product, edge_feat_orders, edge_feat_rev,
           W_user, b_user, W_item, b_item,
           W1_o, b1_o, W2_o, b2_o, W1_r, b1_r, W2_r, b2_r,
           edge_index_orders, edge_index_rev, pos_edge_index, neg_edge_index):
    hc, hp = _proj(h_customer, h_product, W_user, b_user, W_item, b_item)

    src_all = jnp.concatenate([_pad1(edge_index_orders[0], E_PAD),
                               _pad1(edge_index_rev[0], E_PAD)])
    dst_all = jnp.concatenate([_pad1(edge_index_orders[1], E_PAD),
                               _pad1(edge_index_rev[1], E_PAD)])

    src_rows = _gather_src(hc, hp, src_all)

    pad_ef = jnp.zeros((E_PAD - E, D_E), jnp.float32)
    ef_all = jnp.concatenate([edge_feat_orders, pad_ef,
                              edge_feat_rev, pad_ef])
    col = jnp.arange(D_EH * D_H, dtype=jnp.int32)[None, :]
    EK = (col // D_H == jnp.arange(D_EH, dtype=jnp.int32)[:, None]
          ).astype(jnp.float32)
    EH = (col % D_H == jnp.arange(D_H, dtype=jnp.int32)[:, None]
          ).astype(jnp.float32)
    W1Ks = jnp.stack([W1_o @ EK, W1_r @ EK])
    b1rs = jnp.stack([jnp.repeat(b1_o, D_H).reshape(1, D_EH * D_H),
                      jnp.repeat(b1_r, D_H).reshape(1, D_EH * D_H)])
    W2ps = jnp.stack([W2_o.reshape(D_EH * D_H, D_OUT),
                      W2_r.reshape(D_EH * D_H, D_OUT)])
    B2s = jnp.stack([b2_o.reshape(D_H, D_OUT), b2_r.reshape(D_H, D_OUT)])

    msg_all = _edge(ef_all, src_rows, W1Ks, b1rs, W2ps, B2s,
                    EH.astype(jnp.bfloat16))

    zeros_nd = jnp.zeros((N_PAD, DS), jnp.float32)
    agg_o, agg_r = _scatter(msg_all, dst_all, zeros_nd)

    hc_new, hp_new = _finish(agg_r, agg_o)

    idx4 = jnp.concatenate([_pad1(pos_edge_index[0], P_PAD),
                            _pad1(pos_edge_index[1], P_PAD),
                            _pad1(neg_edge_index[0], P_PAD),
                            _pad1(neg_edge_index[1], P_PAD)])
    rows4 = _gather_cos(hc_new, hp_new, idx4).reshape(4, P_PAD, D_OUT)
    pos_score, neg_score = _cos(rows4)

    return hc_new, hp_new, pos_score.reshape(P), neg_score.reshape(P)
